# baseline (speedup 1.0000x reference)
import jax
import jax.numpy as jnp
from jax import lax
from jax.experimental import pallas as pl
from jax.experimental.pallas import tpu as pltpu
from jax.experimental.pallas import tpu_sc as plsc

N_NODES = 10000
N_PAD = 10240
N_EDGES = 320000
IN_DIM = 7
EDGE_DIM = 4
HID = 64
LOC_BIAS = 3.0
EPS = 1e-5

NC = 2
NS = 16
NW = NC * NS
E_PER_W = N_EDGES // NW
CHUNK = 80
NCHUNK = E_PER_W // CHUNK
CROWS = CHUNK // 2
ROWS_PER_TILE = N_PAD // NS
NBUF = 4

EROW = N_EDGES // 2
ROW_PER_W = E_PER_W // 2

F32 = jnp.float32


def _ln(p):
    mu = jnp.mean(p, axis=-1, keepdims=True)
    d = p - mu
    var = jnp.mean(d * d, axis=-1, keepdims=True)
    return d / jnp.sqrt(var + EPS)


def _node_pre_body(x_ref, wn_ref, ws_ref, bs_ref, xwn_ref, hpre_ref):
    x = x_ref[...]
    xwn_ref[...] = jnp.dot(x, wn_ref[...], preferred_element_type=F32)
    hpre_ref[...] = jnp.dot(x, ws_ref[...], preferred_element_type=F32) + bs_ref[...]


def _node_mid_body(agg_ref, hpre_ref, wsrc_ref, wtgt_ref, fb_ref, a_ref, b_ref):
    agg = agg_ref[0, :N_NODES, :] + agg_ref[1, :N_NODES, :]
    h = jnp.maximum(agg + hpre_ref[...], 0.0)
    a_ref[...] = _ln(jnp.dot(h, wsrc_ref[...], preferred_element_type=F32)) + fb_ref[...]
    b_ref[...] = _ln(jnp.dot(h, wtgt_ref[...], preferred_element_type=F32))


def _gate_body(m_ref, s_ref, wm_ref, wo_ref, bo_ref, out_ref):
    p = jnp.dot(m_ref[...], wm_ref[...], preferred_element_type=F32)
    ln = jnp.concatenate([_ln(p[:, :HID]), _ln(p[:, HID:])], axis=1)
    gate = jnp.maximum(s_ref[...] + ln, 0.0)
    logits = jnp.dot(gate, wo_ref[...], preferred_element_type=F32) + bo_ref[...]
    mask = (logits + LOC_BIAS > 0.0).astype(F32)
    out_ref[...] = mask.T


def _sc_msg_body(xwn_hbm, ea_hbm, we_hbm, bm_hbm, src_hbm, dst_hbm, zeros_hbm,
                 m_hbm, agg_hbm,
                 sidx_v, didx_v, ea_v, g_v, w_v, bm_v,
                 agg_sh, sem_in, sem_out, sem_sc):
    c = lax.axis_index("c")
    s = lax.axis_index("s")
    wid = s * NC + c
    ebase = wid * E_PER_W
    mrow = (wid % 16) * E_PER_W
    hoff = (wid // 16) * HID

    pltpu.sync_copy(we_hbm, w_v)
    pltpu.sync_copy(bm_hbm, bm_v)
    pltpu.sync_copy(src_hbm.at[pl.ds(ebase, E_PER_W)], sidx_v)
    pltpu.sync_copy(dst_hbm.at[pl.ds(ebase, E_PER_W)], didx_v)

    pltpu.sync_copy(
        zeros_hbm.at[pl.ds(s * ROWS_PER_TILE, ROWS_PER_TILE)],
        agg_sh.at[pl.ds(s * ROWS_PER_TILE, ROWS_PER_TILE)],
    )
    plsc.subcore_barrier()

    wv = [[w_v[k, pl.ds(q * 16, 16)] for q in range(4)] for k in range(4)]
    bm = [bm_v[pl.ds(q * 16, 16)] for q in range(4)]

    def in_copies(j, b):
        cps = [pltpu.make_async_copy(
                   xwn_hbm.at[sidx_v.at[pl.ds(j * CHUNK, CHUNK)]],
                   g_v.at[b], sem_in.at[b])]
        for k in range(EDGE_DIM):
            cps.append(pltpu.make_async_copy(
                ea_hbm.at[pl.ds(k * N_EDGES + ebase + j * CHUNK, CHUNK)],
                ea_v.at[b, k], sem_in.at[b]))
        return cps

    def out_copy(j, b):
        return pltpu.make_async_copy(
            g_v.at[b],
            m_hbm.at[pl.ds(mrow + j * CHUNK, CHUNK), pl.ds(hoff, HID)],
            sem_out.at[b])

    def issue(j, b):
        for cp in in_copies(j, b):
            cp.start()

    def process(j, b):
        for cp in in_copies(j, b):
            cp.wait()

        def group(g, carry2):
            av = [ea_v[b, k, pl.ds(g * 16, 16)]
                  for k in range(EDGE_DIM)]
            for i in range(16):
                r = g * 16 + i
                for q in range(4):
                    acc = g_v[b, r, pl.ds(q * 16, 16)] + bm[q]
                    for k in range(EDGE_DIM):
                        acc = acc + av[k][i] * wv[k][q]
                    acc = jnp.maximum(acc, 0.0)
                    g_v[b, r, pl.ds(q * 16, 16)] = acc
            return carry2

        lax.fori_loop(0, CHUNK // 16, group, 0)
        out_copy(j, b).start()
        pltpu.async_copy(g_v.at[b],
                         agg_sh.at[didx_v.at[pl.ds(j * CHUNK, CHUNK)]],
                         sem_sc.at[b], add=True)

    for b in range(NBUF):
        issue(b, b)

    def grp(p, carry):
        for b in range(NBUF):
            j = NBUF * p + b
            process(j, b)

            @pl.when(j + NBUF < NCHUNK)
            def _():
                out_copy(j, b).wait()
                pltpu.make_async_copy(
                    g_v.at[b], agg_sh.at[didx_v.at[pl.ds(j * CHUNK, CHUNK)]],
                    sem_sc.at[b]).wait()
                issue(j + NBUF, b)

        return carry

    lax.fori_loop(0, NCHUNK // NBUF, grp, 0)
    for b in range(NCHUNK % NBUF):
        j = NBUF * (NCHUNK // NBUF) + b
        process(j, b)
    for k in range(NBUF):
        j = NCHUNK - NBUF + k
        out_copy(j, j % NBUF).wait()
        pltpu.make_async_copy(
            g_v.at[j % NBUF], agg_sh.at[didx_v.at[pl.ds(j * CHUNK, CHUNK)]],
            sem_sc.at[j % NBUF]).wait()
    plsc.subcore_barrier()

    pltpu.sync_copy(
        agg_sh.at[pl.ds(s * ROWS_PER_TILE, ROWS_PER_TILE)],
        agg_hbm.at[pl.ds(c * N_PAD + s * ROWS_PER_TILE, ROWS_PER_TILE)],
    )


def _sc_pair_body(a_hbm, b_hbm, src_hbm, dst_hbm,
                  s_hbm,
                  sidx_v, didx_v, av, bv, sem_in, sem_out):
    c = lax.axis_index("c")
    s = lax.axis_index("s")
    wid = s * NC + c
    ebase = wid * E_PER_W
    mrow = (wid % 16) * E_PER_W
    hoff = (wid // 16) * HID

    pltpu.sync_copy(src_hbm.at[pl.ds(ebase, E_PER_W)], sidx_v)
    pltpu.sync_copy(dst_hbm.at[pl.ds(ebase, E_PER_W)], didx_v)

    def in_copies(j, b):
        sl = pl.ds(j * CHUNK, CHUNK)
        return (pltpu.make_async_copy(a_hbm.at[sidx_v.at[sl]], av.at[b],
                                      sem_in.at[b]),
                pltpu.make_async_copy(b_hbm.at[didx_v.at[sl]], bv.at[b],
                                      sem_in.at[b]))

    def out_copy(j, b):
        return pltpu.make_async_copy(
            av.at[b],
            s_hbm.at[pl.ds(mrow + j * CHUNK, CHUNK), pl.ds(hoff, HID)],
            sem_out.at[b])

    def issue(j, b):
        for cp in in_copies(j, b):
            cp.start()

    def process(j, b):
        for cp in in_copies(j, b):
            cp.wait()

        def row(r, carry2):
            for q in range(4):
                sl = pl.ds(q * 16, 16)
                av[b, r, sl] = av[b, r, sl] + bv[b, r, sl]
            return carry2

        lax.fori_loop(0, CHUNK, row, 0)
        out_copy(j, b).start()

    for b in range(NBUF):
        issue(b, b)

    def grp(p, carry):
        for b in range(NBUF):
            j = NBUF * p + b
            process(j, b)

            @pl.when(j + NBUF < NCHUNK)
            def _():
                out_copy(j, b).wait()
                issue(j + NBUF, b)

        return carry

    lax.fori_loop(0, NCHUNK // NBUF, grp, 0)
    for b in range(NCHUNK % NBUF):
        j = NBUF * (NCHUNK // NBUF) + b
        process(j, b)
    for k in range(NBUF):
        j = NCHUNK - NBUF + k
        out_copy(j, j % NBUF).wait()


_MESH = dict(core_axis_name="c", subcore_axis_name="s", num_cores=NC,
             num_subcores=NS)

EBLK = 6400


def kernel(x, edge_index, edge_attr, W_node, W_edge, b_msg, W_self, b_self,
           W_src, W_msgp, W_tgt, full_bias, W_out, b_out):
    src1 = edge_index[0]
    dst1 = edge_index[1]
    zeros = jnp.zeros((N_PAD, HID), F32)
    eye2 = jnp.eye(2, dtype=F32)
    w_msgp2 = jnp.kron(eye2, W_msgp)
    w_out2 = jnp.kron(eye2, W_out)
    b_out2 = jnp.tile(b_out, 2).reshape(1, 2)

    xwn, hpre = pl.pallas_call(
        _node_pre_body,
        out_shape=[jax.ShapeDtypeStruct((N_NODES, HID), F32),
                   jax.ShapeDtypeStruct((N_NODES, HID), F32)],
    )(x, W_node, W_self, b_self.reshape(1, HID))

    m2, agg = pl.kernel(
        _sc_msg_body,
        out_type=[jax.ShapeDtypeStruct((EROW, 2 * HID), F32),
                  jax.ShapeDtypeStruct((NC * N_PAD, HID), F32)],
        mesh=plsc.VectorSubcoreMesh(**_MESH),
        compiler_params=pltpu.CompilerParams(use_tc_tiling_on_sc=False),
        scratch_types=[pltpu.VMEM((E_PER_W,), jnp.int32),
                       pltpu.VMEM((E_PER_W,), jnp.int32),
                       pltpu.VMEM((NBUF, EDGE_DIM, CHUNK), F32),
                       pltpu.VMEM((NBUF, CHUNK, HID), F32),
                       pltpu.VMEM((EDGE_DIM, HID), F32),
                       pltpu.VMEM((HID,), F32),
                       pltpu.VMEM_SHARED((N_PAD, HID), F32),
                       pltpu.SemaphoreType.DMA((NBUF,)),
                       pltpu.SemaphoreType.DMA((NBUF,)),
                       pltpu.SemaphoreType.DMA((NBUF,))],
    )(xwn, jnp.transpose(edge_attr).reshape(-1), W_edge, b_msg, src1, dst1,
      zeros)

    a_t, b_t = pl.pallas_call(
        _node_mid_body,
        out_shape=[jax.ShapeDtypeStruct((N_NODES, HID), F32),
                   jax.ShapeDtypeStruct((N_NODES, HID), F32)],
    )(agg.reshape(NC, N_PAD, HID), hpre, W_src, W_tgt,
      full_bias.reshape(1, HID))

    s_e = pl.kernel(
        _sc_pair_body,
        out_type=jax.ShapeDtypeStruct((EROW, 2 * HID), F32),
        mesh=plsc.VectorSubcoreMesh(**_MESH),
        compiler_params=pltpu.CompilerParams(use_tc_tiling_on_sc=False),
        scratch_types=[pltpu.VMEM((E_PER_W,), jnp.int32),
                       pltpu.VMEM((E_PER_W,), jnp.int32),
                       pltpu.VMEM((NBUF, CHUNK, HID), F32),
                       pltpu.VMEM((NBUF, CHUNK, HID), F32),
                       pltpu.SemaphoreType.DMA((NBUF,)),
                       pltpu.SemaphoreType.DMA((NBUF,))],
    )(a_t, b_t, src1, dst1)

    mask2 = pl.pallas_call(
        _gate_body,
        grid=(EROW // EBLK,),
        in_specs=[pl.BlockSpec((EBLK, 2 * HID), lambda i: (i, 0)),
                  pl.BlockSpec((EBLK, 2 * HID), lambda i: (i, 0)),
                  pl.BlockSpec((2 * HID, 2 * HID), lambda i: (0, 0)),
                  pl.BlockSpec((2 * HID, 2), lambda i: (0, 0)),
                  pl.BlockSpec((1, 2), lambda i: (0, 0))],
        out_specs=pl.BlockSpec((2, EBLK), lambda i: (0, i)),
        out_shape=jax.ShapeDtypeStruct((2, EROW), F32),
    )(m2, s_e, w_msgp2, w_out2, b_out2)

    penalty = jnp.zeros((), F32)
    return mask2.reshape(N_EDGES), penalty


# speedup vs baseline: 1.1326x; 1.1326x over previous
import jax
import jax.numpy as jnp
from jax import lax
from jax.experimental import pallas as pl
from jax.experimental.pallas import tpu as pltpu
from jax.experimental.pallas import tpu_sc as plsc

N_NODES = 10000
N_PAD = 10240
N_EDGES = 320000
IN_DIM = 7
EDGE_DIM = 4
HID = 64
LOC_BIAS = 3.0
EPS = 1e-5

NC = 2
NS = 16
NW = NC * NS
E_PER_W = N_EDGES // NW
CHUNK = 80
NCHUNK = E_PER_W // CHUNK
CROWS = CHUNK // 2
ROWS_PER_TILE = N_PAD // NS
NBUF = 4

EROW = N_EDGES // 2
ROW_PER_W = E_PER_W // 2

F32 = jnp.float32


def _ln(p):
    mu = jnp.mean(p, axis=-1, keepdims=True)
    d = p - mu
    var = jnp.mean(d * d, axis=-1, keepdims=True)
    return d / jnp.sqrt(var + EPS)


def _node_pre_body(x_ref, wn_ref, ws_ref, bs_ref, xwn_ref, hpre_ref):
    x = x_ref[...]
    xwn_ref[...] = jnp.dot(x, wn_ref[...], preferred_element_type=F32)
    hpre_ref[...] = jnp.dot(x, ws_ref[...], preferred_element_type=F32) + bs_ref[...]


def _node_mid_body(agg_ref, hpre_ref, wsrc_ref, wtgt_ref, fb_ref, a_ref, b_ref):
    agg = agg_ref[0, :N_NODES, :] + agg_ref[1, :N_NODES, :]
    h = jnp.maximum(agg + hpre_ref[...], 0.0)
    a_ref[...] = _ln(jnp.dot(h, wsrc_ref[...], preferred_element_type=F32)) + fb_ref[...]
    b_ref[...] = _ln(jnp.dot(h, wtgt_ref[...], preferred_element_type=F32))


def _gate_body(m_ref, s_ref, wm_ref, wo_ref, bo_ref, out_ref):
    p = jnp.dot(m_ref[...], wm_ref[...], preferred_element_type=F32)
    ln = jnp.concatenate([_ln(p[:, :HID]), _ln(p[:, HID:])], axis=1)
    gate = jnp.maximum(s_ref[...] + ln, 0.0)
    logits = jnp.dot(gate, wo_ref[...], preferred_element_type=F32) + bo_ref[...]
    mask = (logits + LOC_BIAS > 0.0).astype(F32)
    out_ref[...] = mask.T


def _sc_msg_body(xwn_hbm, ea_hbm, we_hbm, bm_hbm, src_hbm, dst_hbm, zeros_hbm,
                 m_hbm, agg_hbm,
                 sidx_v, didx_v, ea_v, g_v, m128_v, w_v, bm_v,
                 agg_sh, sem_in, sem_out, sem_sc):
    c = lax.axis_index("c")
    s = lax.axis_index("s")
    wid = s * NC + c
    ebase = wid * E_PER_W
    rbase = wid * ROW_PER_W

    pltpu.sync_copy(we_hbm, w_v)
    pltpu.sync_copy(bm_hbm, bm_v)
    pltpu.sync_copy(src_hbm.at[pl.ds(ebase, E_PER_W)], sidx_v)
    pltpu.sync_copy(dst_hbm.at[pl.ds(ebase, E_PER_W)], didx_v)

    pltpu.sync_copy(
        zeros_hbm.at[pl.ds(s * ROWS_PER_TILE, ROWS_PER_TILE)],
        agg_sh.at[pl.ds(s * ROWS_PER_TILE, ROWS_PER_TILE)],
    )
    plsc.subcore_barrier()

    wv = [[w_v[k, pl.ds(q * 16, 16)] for q in range(4)] for k in range(4)]
    bm = [bm_v[pl.ds(q * 16, 16)] for q in range(4)]

    def in_copies(j, b):
        cps = [pltpu.make_async_copy(
                   xwn_hbm.at[sidx_v.at[pl.ds(j * CHUNK, CHUNK)]],
                   g_v.at[b], sem_in.at[b])]
        for k in range(EDGE_DIM):
            cps.append(pltpu.make_async_copy(
                ea_hbm.at[pl.ds(k * N_EDGES + ebase + j * CHUNK, CHUNK)],
                ea_v.at[b, k], sem_in.at[b]))
        return cps

    def out_copy(j, b):
        return pltpu.make_async_copy(
            m128_v.at[b], m_hbm.at[pl.ds(rbase + j * CROWS, CROWS)],
            sem_out.at[b])

    def issue(j, b):
        for cp in in_copies(j, b):
            cp.start()

    def process(j, b):
        for cp in in_copies(j, b):
            cp.wait()

        def group(g, carry2):
            av = [ea_v[b, k, pl.ds(g * 16, 16)]
                  for k in range(EDGE_DIM)]
            for i in range(16):
                r = g * 16 + i
                for q in range(4):
                    acc = g_v[b, r, pl.ds(q * 16, 16)] + bm[q]
                    for k in range(EDGE_DIM):
                        acc = acc + av[k][i] * wv[k][q]
                    acc = jnp.maximum(acc, 0.0)
                    g_v[b, r, pl.ds(q * 16, 16)] = acc
                    m128_v[b, g * 8 + i // 2,
                           pl.ds((4 * (i % 2) + q) * 16, 16)] = acc
            return carry2

        lax.fori_loop(0, CHUNK // 16, group, 0)
        out_copy(j, b).start()
        pltpu.async_copy(g_v.at[b],
                         agg_sh.at[didx_v.at[pl.ds(j * CHUNK, CHUNK)]],
                         sem_sc.at[b], add=True)

    for b in range(NBUF):
        issue(b, b)

    def grp(p, carry):
        for b in range(NBUF):
            j = NBUF * p + b
            process(j, b)

            @pl.when(j + NBUF < NCHUNK)
            def _():
                out_copy(j, b).wait()
                pltpu.make_async_copy(
                    g_v.at[b], agg_sh.at[didx_v.at[pl.ds(j * CHUNK, CHUNK)]],
                    sem_sc.at[b]).wait()
                issue(j + NBUF, b)

        return carry

    lax.fori_loop(0, NCHUNK // NBUF, grp, 0)
    for b in range(NCHUNK % NBUF):
        j = NBUF * (NCHUNK // NBUF) + b
        process(j, b)
    for k in range(NBUF):
        j = NCHUNK - NBUF + k
        out_copy(j, j % NBUF).wait()
        pltpu.make_async_copy(
            g_v.at[j % NBUF], agg_sh.at[didx_v.at[pl.ds(j * CHUNK, CHUNK)]],
            sem_sc.at[j % NBUF]).wait()
    plsc.subcore_barrier()

    pltpu.sync_copy(
        agg_sh.at[pl.ds(s * ROWS_PER_TILE, ROWS_PER_TILE)],
        agg_hbm.at[pl.ds(c * N_PAD + s * ROWS_PER_TILE, ROWS_PER_TILE)],
    )


def _sc_pair_body(a_hbm, b_hbm, src_hbm, dst_hbm,
                  s_hbm,
                  sidx_v, didx_v, av, bv, sv, sem_in, sem_out):
    c = lax.axis_index("c")
    s = lax.axis_index("s")
    wid = s * NC + c
    ebase = wid * E_PER_W
    rbase = wid * ROW_PER_W

    pltpu.sync_copy(src_hbm.at[pl.ds(ebase, E_PER_W)], sidx_v)
    pltpu.sync_copy(dst_hbm.at[pl.ds(ebase, E_PER_W)], didx_v)

    def in_copies(j, b):
        sl = pl.ds(j * CHUNK, CHUNK)
        return (pltpu.make_async_copy(a_hbm.at[sidx_v.at[sl]], av.at[b],
                                      sem_in.at[b]),
                pltpu.make_async_copy(b_hbm.at[didx_v.at[sl]], bv.at[b],
                                      sem_in.at[b]))

    def out_copy(j, b):
        return pltpu.make_async_copy(
            sv.at[b], s_hbm.at[pl.ds(rbase + j * CROWS, CROWS)],
            sem_out.at[b])

    def issue(j, b):
        for cp in in_copies(j, b):
            cp.start()

    def process(j, b):
        for cp in in_copies(j, b):
            cp.wait()

        def rowpair(rp, carry2):
            for half in range(2):
                r = 2 * rp + half
                for q in range(4):
                    sl = pl.ds(q * 16, 16)
                    sv[b, rp, pl.ds((4 * half + q) * 16, 16)] = (
                        av[b, r, sl] + bv[b, r, sl])
            return carry2

        lax.fori_loop(0, CROWS, rowpair, 0)
        out_copy(j, b).start()

    for b in range(NBUF):
        issue(b, b)

    def grp(p, carry):
        for b in range(NBUF):
            j = NBUF * p + b
            process(j, b)

            @pl.when(j + NBUF < NCHUNK)
            def _():
                out_copy(j, b).wait()
                issue(j + NBUF, b)

        return carry

    lax.fori_loop(0, NCHUNK // NBUF, grp, 0)
    for b in range(NCHUNK % NBUF):
        j = NBUF * (NCHUNK // NBUF) + b
        process(j, b)
    for k in range(NBUF):
        j = NCHUNK - NBUF + k
        out_copy(j, j % NBUF).wait()


_MESH = dict(core_axis_name="c", subcore_axis_name="s", num_cores=NC,
             num_subcores=NS)

EBLK = 6400


def kernel(x, edge_index, edge_attr, W_node, W_edge, b_msg, W_self, b_self,
           W_src, W_msgp, W_tgt, full_bias, W_out, b_out):
    src1 = edge_index[0]
    dst1 = edge_index[1]
    zeros = jnp.zeros((N_PAD, HID), F32)
    eye2 = jnp.eye(2, dtype=F32)
    w_msgp2 = jnp.kron(eye2, W_msgp)
    w_out2 = jnp.kron(eye2, W_out)
    b_out2 = jnp.tile(b_out, 2).reshape(1, 2)

    xwn, hpre = pl.pallas_call(
        _node_pre_body,
        out_shape=[jax.ShapeDtypeStruct((N_NODES, HID), F32),
                   jax.ShapeDtypeStruct((N_NODES, HID), F32)],
    )(x, W_node, W_self, b_self.reshape(1, HID))

    m2, agg = pl.kernel(
        _sc_msg_body,
        out_type=[jax.ShapeDtypeStruct((EROW, 2 * HID), F32),
                  jax.ShapeDtypeStruct((NC * N_PAD, HID), F32)],
        mesh=plsc.VectorSubcoreMesh(**_MESH),
        compiler_params=pltpu.CompilerParams(use_tc_tiling_on_sc=False),
        scratch_types=[pltpu.VMEM((E_PER_W,), jnp.int32),
                       pltpu.VMEM((E_PER_W,), jnp.int32),
                       pltpu.VMEM((NBUF, EDGE_DIM, CHUNK), F32),
                       pltpu.VMEM((NBUF, CHUNK, HID), F32),
                       pltpu.VMEM((NBUF, CROWS, 2 * HID), F32),
                       pltpu.VMEM((EDGE_DIM, HID), F32),
                       pltpu.VMEM((HID,), F32),
                       pltpu.VMEM_SHARED((N_PAD, HID), F32),
                       pltpu.SemaphoreType.DMA((NBUF,)),
                       pltpu.SemaphoreType.DMA((NBUF,)),
                       pltpu.SemaphoreType.DMA((NBUF,))],
    )(xwn, jnp.transpose(edge_attr).reshape(-1), W_edge, b_msg, src1, dst1,
      zeros)

    a_t, b_t = pl.pallas_call(
        _node_mid_body,
        out_shape=[jax.ShapeDtypeStruct((N_NODES, HID), F32),
                   jax.ShapeDtypeStruct((N_NODES, HID), F32)],
    )(agg.reshape(NC, N_PAD, HID), hpre, W_src, W_tgt,
      full_bias.reshape(1, HID))

    s_e = pl.kernel(
        _sc_pair_body,
        out_type=jax.ShapeDtypeStruct((EROW, 2 * HID), F32),
        mesh=plsc.VectorSubcoreMesh(**_MESH),
        compiler_params=pltpu.CompilerParams(use_tc_tiling_on_sc=False),
        scratch_types=[pltpu.VMEM((E_PER_W,), jnp.int32),
                       pltpu.VMEM((E_PER_W,), jnp.int32),
                       pltpu.VMEM((NBUF, CHUNK, HID), F32),
                       pltpu.VMEM((NBUF, CHUNK, HID), F32),
                       pltpu.VMEM((NBUF, CROWS, 2 * HID), F32),
                       pltpu.SemaphoreType.DMA((NBUF,)),
                       pltpu.SemaphoreType.DMA((NBUF,))],
    )(a_t, b_t, src1, dst1)

    mask2 = pl.pallas_call(
        _gate_body,
        grid=(EROW // EBLK,),
        in_specs=[pl.BlockSpec((EBLK, 2 * HID), lambda i: (i, 0)),
                  pl.BlockSpec((EBLK, 2 * HID), lambda i: (i, 0)),
                  pl.BlockSpec((2 * HID, 2 * HID), lambda i: (0, 0)),
                  pl.BlockSpec((2 * HID, 2), lambda i: (0, 0)),
                  pl.BlockSpec((1, 2), lambda i: (0, 0))],
        out_specs=pl.BlockSpec((2, EBLK), lambda i: (0, i)),
        out_shape=jax.ShapeDtypeStruct((2, EROW), F32),
    )(m2, s_e, w_msgp2, w_out2, b_out2)

    penalty = jnp.zeros((), F32)
    return mask2.T.reshape(N_EDGES), penalty


# speedup vs baseline: 1.2573x; 1.1101x over previous
import jax
import jax.numpy as jnp
from jax import lax
from jax.experimental import pallas as pl
from jax.experimental.pallas import tpu as pltpu
from jax.experimental.pallas import tpu_sc as plsc

N_NODES = 10000
N_PAD = 10240
N_EDGES = 320000
IN_DIM = 7
EDGE_DIM = 4
HID = 64
LOC_BIAS = 3.0
EPS = 1e-5

NC = 2
NS = 16
NW = NC * NS
E_PER_W = N_EDGES // NW
CHUNK = 80
NCHUNK = E_PER_W // CHUNK
CROWS = CHUNK // 2
ROWS_PER_TILE = N_PAD // NS
NBUF = 4

EROW = N_EDGES // 2
ROW_PER_W = E_PER_W // 2

F32 = jnp.float32


def _ln(p):
    mu = jnp.mean(p, axis=-1, keepdims=True)
    d = p - mu
    var = jnp.mean(d * d, axis=-1, keepdims=True)
    return d / jnp.sqrt(var + EPS)


def _node_pre_body(x_ref, wn_ref, ws_ref, bs_ref, xwn_ref, hpre_ref):
    x = x_ref[...]
    xwn_ref[...] = jnp.dot(x, wn_ref[...], preferred_element_type=F32)
    hpre_ref[...] = jnp.dot(x, ws_ref[...], preferred_element_type=F32) + bs_ref[...]


def _node_mid_body(agg_ref, hpre_ref, wsrc_ref, wtgt_ref, fb_ref, a_ref, b_ref):
    agg = agg_ref[0, :N_NODES, :] + agg_ref[1, :N_NODES, :]
    h = jnp.maximum(agg + hpre_ref[...], 0.0)
    a_ref[...] = _ln(jnp.dot(h, wsrc_ref[...], preferred_element_type=F32)) + fb_ref[...]
    b_ref[...] = _ln(jnp.dot(h, wtgt_ref[...], preferred_element_type=F32))


def _lnp_body(m_ref, wm_ref, out_ref):
    p = jnp.dot(m_ref[...], wm_ref[...], preferred_element_type=F32)
    out_ref[...] = jnp.concatenate([_ln(p[:, :HID]), _ln(p[:, HID:])], axis=1)


def _gate_body(lnp_ref, s_ref, wo_ref, bo_ref, out_ref):
    gate = jnp.maximum(s_ref[...] + lnp_ref[...], 0.0)
    logits = jnp.dot(gate, wo_ref[...], preferred_element_type=F32) + bo_ref[...]
    mask = (logits + LOC_BIAS > 0.0).astype(F32)
    out_ref[...] = mask.T


def _sc_msg_body(xwn_hbm, ea_hbm, we_hbm, bm_hbm, src_hbm, dst_hbm, zeros_hbm,
                 m_hbm, agg_hbm,
                 sidx_v, didx_v, ea_v, g_v, m128_v, w_v, bm_v,
                 agg_sh, sem_in, sem_out, sem_sc):
    c = lax.axis_index("c")
    s = lax.axis_index("s")
    wid = s * NC + c
    ebase = wid * E_PER_W
    rbase = wid * ROW_PER_W

    pltpu.sync_copy(we_hbm, w_v)
    pltpu.sync_copy(bm_hbm, bm_v)
    pltpu.sync_copy(src_hbm.at[pl.ds(ebase, E_PER_W)], sidx_v)
    pltpu.sync_copy(dst_hbm.at[pl.ds(ebase, E_PER_W)], didx_v)

    pltpu.sync_copy(
        zeros_hbm.at[pl.ds(s * ROWS_PER_TILE, ROWS_PER_TILE)],
        agg_sh.at[pl.ds(s * ROWS_PER_TILE, ROWS_PER_TILE)],
    )
    plsc.subcore_barrier()

    wv = [[w_v[k, pl.ds(q * 16, 16)] for q in range(4)] for k in range(4)]
    bm = [bm_v[pl.ds(q * 16, 16)] for q in range(4)]

    def in_copies(j, b):
        cps = [pltpu.make_async_copy(
                   xwn_hbm.at[sidx_v.at[pl.ds(j * CHUNK, CHUNK)]],
                   g_v.at[b], sem_in.at[b])]
        for k in range(EDGE_DIM):
            cps.append(pltpu.make_async_copy(
                ea_hbm.at[pl.ds(k * N_EDGES + ebase + j * CHUNK, CHUNK)],
                ea_v.at[b, k], sem_in.at[b]))
        return cps

    def out_copy(j, b):
        return pltpu.make_async_copy(
            m128_v.at[b], m_hbm.at[pl.ds(rbase + j * CROWS, CROWS)],
            sem_out.at[b])

    def issue(j, b):
        for cp in in_copies(j, b):
            cp.start()

    def process(j, b):
        for cp in in_copies(j, b):
            cp.wait()

        def group(g, carry2):
            av = [ea_v[b, k, pl.ds(g * 16, 16)]
                  for k in range(EDGE_DIM)]
            for i in range(16):
                r = g * 16 + i
                for q in range(4):
                    acc = g_v[b, r, pl.ds(q * 16, 16)] + bm[q]
                    for k in range(EDGE_DIM):
                        acc = acc + av[k][i] * wv[k][q]
                    acc = jnp.maximum(acc, 0.0)
                    g_v[b, r, pl.ds(q * 16, 16)] = acc
                    m128_v[b, g * 8 + i // 2,
                           pl.ds((4 * (i % 2) + q) * 16, 16)] = acc
            return carry2

        lax.fori_loop(0, CHUNK // 16, group, 0)
        out_copy(j, b).start()
        pltpu.async_copy(g_v.at[b],
                         agg_sh.at[didx_v.at[pl.ds(j * CHUNK, CHUNK)]],
                         sem_sc.at[b], add=True)

    for b in range(NBUF):
        issue(b, b)

    def grp(p, carry):
        for b in range(NBUF):
            j = NBUF * p + b
            process(j, b)

            @pl.when(j + NBUF < NCHUNK)
            def _():
                out_copy(j, b).wait()
                pltpu.make_async_copy(
                    g_v.at[b], agg_sh.at[didx_v.at[pl.ds(j * CHUNK, CHUNK)]],
                    sem_sc.at[b]).wait()
                issue(j + NBUF, b)

        return carry

    lax.fori_loop(0, NCHUNK // NBUF, grp, 0)
    for b in range(NCHUNK % NBUF):
        j = NBUF * (NCHUNK // NBUF) + b
        process(j, b)
    for k in range(NBUF):
        j = NCHUNK - NBUF + k
        out_copy(j, j % NBUF).wait()
        pltpu.make_async_copy(
            g_v.at[j % NBUF], agg_sh.at[didx_v.at[pl.ds(j * CHUNK, CHUNK)]],
            sem_sc.at[j % NBUF]).wait()
    plsc.subcore_barrier()

    pltpu.sync_copy(
        agg_sh.at[pl.ds(s * ROWS_PER_TILE, ROWS_PER_TILE)],
        agg_hbm.at[pl.ds(c * N_PAD + s * ROWS_PER_TILE, ROWS_PER_TILE)],
    )


def _sc_pair_body(a_hbm, b_hbm, src_hbm, dst_hbm,
                  s_hbm,
                  sidx_v, didx_v, av, bv, sv, sem_in, sem_out):
    c = lax.axis_index("c")
    s = lax.axis_index("s")
    wid = s * NC + c
    ebase = wid * E_PER_W
    rbase = wid * ROW_PER_W

    pltpu.sync_copy(src_hbm.at[pl.ds(ebase, E_PER_W)], sidx_v)
    pltpu.sync_copy(dst_hbm.at[pl.ds(ebase, E_PER_W)], didx_v)

    def in_copies(j, b):
        sl = pl.ds(j * CHUNK, CHUNK)
        return (pltpu.make_async_copy(a_hbm.at[sidx_v.at[sl]], av.at[b],
                                      sem_in.at[b]),
                pltpu.make_async_copy(b_hbm.at[didx_v.at[sl]], bv.at[b],
                                      sem_in.at[b]))

    def out_copy(j, b):
        return pltpu.make_async_copy(
            sv.at[b], s_hbm.at[pl.ds(rbase + j * CROWS, CROWS)],
            sem_out.at[b])

    def issue(j, b):
        for cp in in_copies(j, b):
            cp.start()

    def process(j, b):
        for cp in in_copies(j, b):
            cp.wait()

        def rowpair(rp, carry2):
            for half in range(2):
                r = 2 * rp + half
                for q in range(4):
                    sl = pl.ds(q * 16, 16)
                    sv[b, rp, pl.ds((4 * half + q) * 16, 16)] = (
                        av[b, r, sl] + bv[b, r, sl])
            return carry2

        lax.fori_loop(0, CROWS, rowpair, 0)
        out_copy(j, b).start()

    for b in range(NBUF):
        issue(b, b)

    def grp(p, carry):
        for b in range(NBUF):
            j = NBUF * p + b
            process(j, b)

            @pl.when(j + NBUF < NCHUNK)
            def _():
                out_copy(j, b).wait()
                issue(j + NBUF, b)

        return carry

    lax.fori_loop(0, NCHUNK // NBUF, grp, 0)
    for b in range(NCHUNK % NBUF):
        j = NBUF * (NCHUNK // NBUF) + b
        process(j, b)
    for k in range(NBUF):
        j = NCHUNK - NBUF + k
        out_copy(j, j % NBUF).wait()


_MESH = dict(core_axis_name="c", subcore_axis_name="s", num_cores=NC,
             num_subcores=NS)

EBLK = 6400


def kernel(x, edge_index, edge_attr, W_node, W_edge, b_msg, W_self, b_self,
           W_src, W_msgp, W_tgt, full_bias, W_out, b_out):
    src1 = edge_index[0]
    dst1 = edge_index[1]
    zeros = jnp.zeros((N_PAD, HID), F32)
    eye2 = jnp.eye(2, dtype=F32)
    w_msgp2 = jnp.kron(eye2, W_msgp)
    w_out2 = jnp.kron(eye2, W_out)
    b_out2 = jnp.tile(b_out, 2).reshape(1, 2)

    xwn, hpre = pl.pallas_call(
        _node_pre_body,
        out_shape=[jax.ShapeDtypeStruct((N_NODES, HID), F32),
                   jax.ShapeDtypeStruct((N_NODES, HID), F32)],
    )(x, W_node, W_self, b_self.reshape(1, HID))

    m2, agg = pl.kernel(
        _sc_msg_body,
        out_type=[jax.ShapeDtypeStruct((EROW, 2 * HID), F32),
                  jax.ShapeDtypeStruct((NC * N_PAD, HID), F32)],
        mesh=plsc.VectorSubcoreMesh(**_MESH),
        compiler_params=pltpu.CompilerParams(use_tc_tiling_on_sc=False),
        scratch_types=[pltpu.VMEM((E_PER_W,), jnp.int32),
                       pltpu.VMEM((E_PER_W,), jnp.int32),
                       pltpu.VMEM((NBUF, EDGE_DIM, CHUNK), F32),
                       pltpu.VMEM((NBUF, CHUNK, HID), F32),
                       pltpu.VMEM((NBUF, CROWS, 2 * HID), F32),
                       pltpu.VMEM((EDGE_DIM, HID), F32),
                       pltpu.VMEM((HID,), F32),
                       pltpu.VMEM_SHARED((N_PAD, HID), F32),
                       pltpu.SemaphoreType.DMA((NBUF,)),
                       pltpu.SemaphoreType.DMA((NBUF,)),
                       pltpu.SemaphoreType.DMA((NBUF,))],
    )(xwn, jnp.transpose(edge_attr).reshape(-1), W_edge, b_msg, src1, dst1,
      zeros)

    a_t, b_t = pl.pallas_call(
        _node_mid_body,
        out_shape=[jax.ShapeDtypeStruct((N_NODES, HID), F32),
                   jax.ShapeDtypeStruct((N_NODES, HID), F32)],
    )(agg.reshape(NC, N_PAD, HID), hpre, W_src, W_tgt,
      full_bias.reshape(1, HID))

    s_e = pl.kernel(
        _sc_pair_body,
        out_type=jax.ShapeDtypeStruct((EROW, 2 * HID), F32),
        mesh=plsc.VectorSubcoreMesh(**_MESH),
        compiler_params=pltpu.CompilerParams(use_tc_tiling_on_sc=False),
        scratch_types=[pltpu.VMEM((E_PER_W,), jnp.int32),
                       pltpu.VMEM((E_PER_W,), jnp.int32),
                       pltpu.VMEM((NBUF, CHUNK, HID), F32),
                       pltpu.VMEM((NBUF, CHUNK, HID), F32),
                       pltpu.VMEM((NBUF, CROWS, 2 * HID), F32),
                       pltpu.SemaphoreType.DMA((NBUF,)),
                       pltpu.SemaphoreType.DMA((NBUF,))],
    )(a_t, b_t, src1, dst1)

    lnp2 = pl.pallas_call(
        _lnp_body,
        grid=(EROW // EBLK,),
        in_specs=[pl.BlockSpec((EBLK, 2 * HID), lambda i: (i, 0)),
                  pl.BlockSpec((2 * HID, 2 * HID), lambda i: (0, 0))],
        out_specs=pl.BlockSpec((EBLK, 2 * HID), lambda i: (i, 0)),
        out_shape=jax.ShapeDtypeStruct((EROW, 2 * HID), F32),
    )(m2, w_msgp2)

    mask2 = pl.pallas_call(
        _gate_body,
        grid=(EROW // EBLK,),
        in_specs=[pl.BlockSpec((EBLK, 2 * HID), lambda i: (i, 0)),
                  pl.BlockSpec((EBLK, 2 * HID), lambda i: (i, 0)),
                  pl.BlockSpec((2 * HID, 2), lambda i: (0, 0)),
                  pl.BlockSpec((1, 2), lambda i: (0, 0))],
        out_specs=pl.BlockSpec((2, EBLK), lambda i: (0, i)),
        out_shape=jax.ShapeDtypeStruct((2, EROW), F32),
    )(lnp2, s_e, w_out2, b_out2)

    penalty = jnp.zeros((), F32)
    return mask2.T.reshape(N_EDGES), penalty
